# baseline (speedup 1.0000x reference)
import functools

import jax
import jax.numpy as jnp
from jax.experimental import pallas as pl
from jax.experimental.pallas import tpu as pltpu
from jax.experimental.pallas import tpu_sc as plsc

N = 16384
C = 20
K = 16
Q = 128
G = N // Q
PC = 32
BIG = 3e38

NW = 32
PTS_W = N // NW
CH = 32
NJ = CH * K // 128
PCG = 128
NR = PTS_W * K // 128


NG = 1024
NS = N // NG


def _knn_kernel(pred_ref, q_ref, cpt_ref, probs_ref, idx_ref):
    logits = pred_ref[...]
    m = jnp.max(logits, axis=1, keepdims=True)
    e = jnp.exp(logits - m)
    p = e / jnp.sum(e, axis=1, keepdims=True)
    probs_ref[...] = jnp.concatenate(
        [p, jnp.zeros((Q, PCG - C), jnp.float32)], axis=1)

    q = q_ref[...]
    cpt = cpt_ref[...]
    d = (q[:, 0:1] - cpt[0:1, :]) ** 2
    d += (q[:, 1:2] - cpt[1:2, :]) ** 2
    d += (q[:, 2:3] - cpt[2:3, :]) ** 2

    gm1 = jnp.full((Q, NG), BIG, jnp.float32)
    gm2 = jnp.full((Q, NG), BIG, jnp.float32)
    sid1 = jnp.zeros((Q, NG), jnp.int32)
    sid2 = jnp.zeros((Q, NG), jnp.int32)
    for s in range(NS):
        v = d[:, s * NG:(s + 1) * NG]
        lt1 = v < gm1
        lt2 = v < gm2
        gm2 = jnp.where(lt1, gm1, jnp.where(lt2, v, gm2))
        sid2 = jnp.where(lt1, sid1, jnp.where(lt2, s, sid2))
        gm1 = jnp.where(lt1, v, gm1)
        sid1 = jnp.where(lt1, s, sid1)

    iota = jax.lax.broadcasted_iota(jnp.int32, (Q, NG), 1)
    cols = []
    for _ in range(K):
        t = jnp.min(gm1, axis=1, keepdims=True)
        sel = gm1 <= t
        jcol = jnp.min(jnp.where(sel, iota, NG), axis=1, keepdims=True)
        onehot = iota == jcol
        s_star = jnp.max(jnp.where(onehot, sid1, 0), axis=1, keepdims=True)
        cols.append(s_star * NG + jcol)
        gm1 = jnp.where(onehot, gm2, gm1)
        sid1 = jnp.where(onehot, sid2, sid1)
        gm2 = jnp.where(onehot, BIG, gm2)
    idx_ref[...] = jnp.concatenate(cols, axis=1)


def _sc_loss_kernel(probs_hbm, idx2_hbm, out_hbm, idx_v, nbr_v, ctr_v, scr,
                    sem):
    nc = 2
    wid = jax.lax.axis_index("s") * nc + jax.lax.axis_index("c")
    w_pt_base = wid * PTS_W

    pltpu.sync_copy(
        idx2_hbm.at[pl.ds(pl.multiple_of(wid * NR, NR), NR)], idx_v)

    def chunk_body(ci, accs):
        a0, a1 = accs
        pt_base = pl.multiple_of(w_pt_base + ci * CH, CH)
        copies = [
            pltpu.async_copy(probs_hbm.at[idx_v.at[ci * NJ + j]],
                             nbr_v.at[j], sem)
            for j in range(NJ)
        ]
        pltpu.sync_copy(probs_hbm.at[pl.ds(pt_base, CH)], ctr_v)
        for cpy in copies:
            cpy.wait()

        def pt_body(p, acc2):
            b0, b1 = acc2
            c0 = ctr_v[p, pl.ds(0, 16)]
            c1 = ctr_v[p, pl.ds(16, 16)]
            j = p // 8
            for k in range(K):
                col = (p % 8) * K + k
                n0 = nbr_v[j, col, pl.ds(0, 16)]
                n1 = nbr_v[j, col, pl.ds(16, 16)]
                e0 = c0 - n0
                e1 = c1 - n1
                b0 = b0 + e0 * e0
                b1 = b1 + e1 * e1
            return b0, b1

        return jax.lax.fori_loop(0, CH, pt_body, (a0, a1))

    z = jnp.zeros((16,), jnp.float32)
    acc0, acc1 = jax.lax.fori_loop(0, PTS_W // CH, chunk_body, (z, z))
    scr[...] = acc0 + acc1
    pltpu.sync_copy(scr, out_hbm.at[pl.ds(pl.multiple_of(wid * 16, 16), 16)])


def _reduce_kernel(part_ref, out_ref):
    out_ref[...] = (jnp.sum(part_ref[...]).reshape(1, 1)
                    * jnp.float32(1.0 / (K * N)))


def kernel(pred, coord, segment, offset):
    del segment, offset
    coord8 = jnp.pad(coord, ((0, 0), (0, 5)))
    cpt = coord8.T
    probs, idx = pl.pallas_call(
        _knn_kernel,
        grid=(G,),
        in_specs=[
            pl.BlockSpec((Q, C), lambda i: (i, 0)),
            pl.BlockSpec((Q, 8), lambda i: (i, 0)),
            pl.BlockSpec((8, N), lambda i: (0, 0)),
        ],
        out_specs=[
            pl.BlockSpec((Q, PCG), lambda i: (i, 0)),
            pl.BlockSpec((Q, K), lambda i: (i, 0)),
        ],
        out_shape=[
            jax.ShapeDtypeStruct((N, PCG), jnp.float32),
            jax.ShapeDtypeStruct((N, K), jnp.int32),
        ],
        compiler_params=pltpu.CompilerParams(
            dimension_semantics=("arbitrary",),
        ),
    )(pred, coord8, cpt)

    idx2 = idx.reshape(N * K // 128, 128)

    mesh = plsc.VectorSubcoreMesh(core_axis_name="c", subcore_axis_name="s")
    partials = pl.kernel(
        _sc_loss_kernel,
        mesh=mesh,
        out_type=jax.ShapeDtypeStruct((NW * 16,), jnp.float32),
        scratch_types=[
            pltpu.VMEM((NR, 128), jnp.int32),
            pltpu.VMEM((NJ, 128, PCG), jnp.float32),
            pltpu.VMEM((CH, PCG), jnp.float32),
            pltpu.VMEM((16,), jnp.float32),
            pltpu.SemaphoreType.DMA,
        ],
    )(probs, idx2)

    out = pl.pallas_call(
        _reduce_kernel,
        out_specs=pl.BlockSpec((1, 1), lambda: (0, 0)),
        out_shape=jax.ShapeDtypeStruct((1, 1), jnp.float32),
    )(partials.reshape(4, 128))
    return out[0, 0]


# speedup vs baseline: 1.0081x; 1.0081x over previous
import functools

import jax
import jax.numpy as jnp
from jax.experimental import pallas as pl
from jax.experimental.pallas import tpu as pltpu
from jax.experimental.pallas import tpu_sc as plsc

N = 16384
C = 20
K = 16
Q = 256
G = N // Q
PC = 32
BIG = 3e38

NW = 32
PTS_W = N // NW
CH = 32
NJ = CH * K // 128
PCG = 128
NR = PTS_W * K // 128


NG = 1024
NS = N // NG


def _knn_kernel(pred_ref, q_ref, cpt_ref, probs_ref, idx_ref):
    logits = pred_ref[...]
    m = jnp.max(logits, axis=1, keepdims=True)
    e = jnp.exp(logits - m)
    p = e / jnp.sum(e, axis=1, keepdims=True)
    probs_ref[...] = jnp.concatenate(
        [p, jnp.zeros((Q, PCG - C), jnp.float32)], axis=1)

    q = q_ref[...]
    cpt = cpt_ref[...]
    d = (q[:, 0:1] - cpt[0:1, :]) ** 2
    d += (q[:, 1:2] - cpt[1:2, :]) ** 2
    d += (q[:, 2:3] - cpt[2:3, :]) ** 2

    gm1 = jnp.full((Q, NG), BIG, jnp.float32)
    gm2 = jnp.full((Q, NG), BIG, jnp.float32)
    sid1 = jnp.zeros((Q, NG), jnp.int32)
    sid2 = jnp.zeros((Q, NG), jnp.int32)
    for s in range(NS):
        v = d[:, s * NG:(s + 1) * NG]
        lt1 = v < gm1
        lt2 = v < gm2
        gm2 = jnp.where(lt1, gm1, jnp.where(lt2, v, gm2))
        sid2 = jnp.where(lt1, sid1, jnp.where(lt2, s, sid2))
        gm1 = jnp.where(lt1, v, gm1)
        sid1 = jnp.where(lt1, s, sid1)

    iota = jax.lax.broadcasted_iota(jnp.int32, (Q, NG), 1)
    cols = []
    for _ in range(K):
        t = jnp.min(gm1, axis=1, keepdims=True)
        sel = gm1 <= t
        jcol = jnp.min(jnp.where(sel, iota, NG), axis=1, keepdims=True)
        onehot = iota == jcol
        s_star = jnp.max(jnp.where(onehot, sid1, 0), axis=1, keepdims=True)
        cols.append(s_star * NG + jcol)
        gm1 = jnp.where(onehot, gm2, gm1)
        sid1 = jnp.where(onehot, sid2, sid1)
        gm2 = jnp.where(onehot, BIG, gm2)
    idx_ref[...] = jnp.concatenate(cols, axis=1)


def _sc_loss_kernel(probs_hbm, idx2_hbm, out_hbm, idx_v, nbr_v, ctr_v, scr,
                    sem):
    nc = 2
    wid = jax.lax.axis_index("s") * nc + jax.lax.axis_index("c")
    w_pt_base = wid * PTS_W

    pltpu.sync_copy(
        idx2_hbm.at[pl.ds(pl.multiple_of(wid * NR, NR), NR)], idx_v)

    def chunk_body(ci, accs):
        a0, a1 = accs
        pt_base = pl.multiple_of(w_pt_base + ci * CH, CH)
        copies = [
            pltpu.async_copy(probs_hbm.at[idx_v.at[ci * NJ + j]],
                             nbr_v.at[j], sem)
            for j in range(NJ)
        ]
        pltpu.sync_copy(probs_hbm.at[pl.ds(pt_base, CH)], ctr_v)
        for cpy in copies:
            cpy.wait()

        def pt_body(p, acc2):
            b0, b1 = acc2
            c0 = ctr_v[p, pl.ds(0, 16)]
            c1 = ctr_v[p, pl.ds(16, 16)]
            j = p // 8
            for k in range(K):
                col = (p % 8) * K + k
                n0 = nbr_v[j, col, pl.ds(0, 16)]
                n1 = nbr_v[j, col, pl.ds(16, 16)]
                e0 = c0 - n0
                e1 = c1 - n1
                b0 = b0 + e0 * e0
                b1 = b1 + e1 * e1
            return b0, b1

        return jax.lax.fori_loop(0, CH, pt_body, (a0, a1))

    z = jnp.zeros((16,), jnp.float32)
    acc0, acc1 = jax.lax.fori_loop(0, PTS_W // CH, chunk_body, (z, z))
    scr[...] = acc0 + acc1
    pltpu.sync_copy(scr, out_hbm.at[pl.ds(pl.multiple_of(wid * 16, 16), 16)])


def _reduce_kernel(part_ref, out_ref):
    out_ref[...] = (jnp.sum(part_ref[...]).reshape(1, 1)
                    * jnp.float32(1.0 / (K * N)))


def kernel(pred, coord, segment, offset):
    del segment, offset
    coord8 = jnp.pad(coord, ((0, 0), (0, 5)))
    cpt = coord8.T
    probs, idx = pl.pallas_call(
        _knn_kernel,
        grid=(G,),
        in_specs=[
            pl.BlockSpec((Q, C), lambda i: (i, 0)),
            pl.BlockSpec((Q, 8), lambda i: (i, 0)),
            pl.BlockSpec((8, N), lambda i: (0, 0)),
        ],
        out_specs=[
            pl.BlockSpec((Q, PCG), lambda i: (i, 0)),
            pl.BlockSpec((Q, K), lambda i: (i, 0)),
        ],
        out_shape=[
            jax.ShapeDtypeStruct((N, PCG), jnp.float32),
            jax.ShapeDtypeStruct((N, K), jnp.int32),
        ],
        compiler_params=pltpu.CompilerParams(
            dimension_semantics=("arbitrary",),
        ),
    )(pred, coord8, cpt)

    idx2 = idx.reshape(N * K // 128, 128)

    mesh = plsc.VectorSubcoreMesh(core_axis_name="c", subcore_axis_name="s")
    partials = pl.kernel(
        _sc_loss_kernel,
        mesh=mesh,
        out_type=jax.ShapeDtypeStruct((NW * 16,), jnp.float32),
        scratch_types=[
            pltpu.VMEM((NR, 128), jnp.int32),
            pltpu.VMEM((NJ, 128, PCG), jnp.float32),
            pltpu.VMEM((CH, PCG), jnp.float32),
            pltpu.VMEM((16,), jnp.float32),
            pltpu.SemaphoreType.DMA,
        ],
    )(probs, idx2)

    out = pl.pallas_call(
        _reduce_kernel,
        out_specs=pl.BlockSpec((1, 1), lambda: (0, 0)),
        out_shape=jax.ShapeDtypeStruct((1, 1), jnp.float32),
    )(partials.reshape(4, 128))
    return out[0, 0]
